# Initial kernel scaffold; baseline (speedup 1.0000x reference)
import functools

import jax
import jax.numpy as jnp
from jax import lax
from jax.experimental import pallas as pl
from jax.experimental.pallas import tpu as pltpu
from jax.experimental.pallas import tpu_sc as plsc

NC = 2
NS = 16
NW = NC * NS

G = 128
QPC = 4
CHUNK = G * QPC


def _make_gather(tot, d):
    per_w = tot // NW
    n_chunks = per_w // CHUNK
    n_desc = per_w // G
    assert per_w % CHUNK == 0 and n_chunks % 2 == 0

    mesh = plsc.VectorSubcoreMesh(core_axis_name="c", subcore_axis_name="s")

    @functools.partial(
        pl.kernel,
        mesh=mesh,
        out_type=jax.ShapeDtypeStruct((tot, d), jnp.float32),
        scratch_types=[
            pltpu.VMEM((n_desc, G), jnp.int32),
            pltpu.VMEM((2, CHUNK, d), jnp.float32),
            pltpu.SemaphoreType.DMA,
            pltpu.SemaphoreType.DMA,
            pltpu.SemaphoreType.DMA,
        ],
    )
    def gather_kernel(idx_hbm, table_hbm, out_hbm, idx_v, rows_v, gsem,
                      ssem0, ssem1):
        wid = lax.axis_index("s") * NC + lax.axis_index("c")
        base = wid * per_w
        ssems = (ssem0, ssem1)

        pltpu.sync_copy(idx_hbm.at[wid], idx_v)

        def store_copy(j, bb):
            return pltpu.make_async_copy(
                rows_v.at[bb],
                out_hbm.at[pl.ds(base + j * CHUNK, CHUNK)],
                ssems[bb],
            )

        def body(t, carry):
            for bb in range(2):
                j = 2 * t + bb
                @pl.when(t > 0)
                def _():
                    store_copy(j - 2, bb).wait()

                copies = []
                for q in range(QPC):
                    copies.append(pltpu.async_copy(
                        table_hbm.at[idx_v.at[j * QPC + q]],
                        rows_v.at[bb].at[pl.ds(q * G, G)],
                        gsem,
                    ))
                for cp in copies:
                    cp.wait()
                store_copy(j, bb).start()
            return carry

        lax.fori_loop(0, n_chunks // 2, body, 0)
        store_copy(n_chunks - 2, 0).wait()
        store_copy(n_chunks - 1, 1).wait()

    return gather_kernel


def kernel(input_ids, attention_mask, table):
    b, s = input_ids.shape
    _, d = table.shape
    tot = b * s
    ids = input_ids.astype(jnp.int32).reshape(NW, tot // (NW * G), G)
    out = _make_gather(tot, d)(ids, table)
    return out.reshape(b, s, d), attention_mask


# speedup vs baseline: 4.2626x; 4.2626x over previous
import functools

import jax
import jax.numpy as jnp
from jax import lax
from jax.experimental import pallas as pl
from jax.experimental.pallas import tpu as pltpu
from jax.experimental.pallas import tpu_sc as plsc

NC = 2
NS = 16
NW = NC * NS

G = 128
QPC = 4
CHUNK = G * QPC


def _make_gather(tot, d):
    per_w = tot // NW
    n_chunks = per_w // CHUNK
    n_desc = per_w // G
    assert per_w % CHUNK == 0 and n_chunks % 2 == 0

    mesh = plsc.VectorSubcoreMesh(core_axis_name="c", subcore_axis_name="s")

    @functools.partial(
        pl.kernel,
        mesh=mesh,
        out_type=jax.ShapeDtypeStruct((tot, d), jnp.float32),
        scratch_types=[
            pltpu.VMEM((n_desc, G), jnp.int32),
            pltpu.VMEM((2, CHUNK, d), jnp.float32),
            pltpu.SemaphoreType.DMA,
            pltpu.SemaphoreType.DMA,
            pltpu.SemaphoreType.DMA,
        ],
        compiler_params=pltpu.CompilerParams(use_tc_tiling_on_sc=False),
    )
    def gather_kernel(idx_hbm, table_hbm, out_hbm, idx_v, rows_v, gsem,
                      ssem0, ssem1):
        wid = lax.axis_index("s") * NC + lax.axis_index("c")
        base = wid * per_w
        ssems = (ssem0, ssem1)

        pltpu.sync_copy(idx_hbm.at[wid], idx_v)

        def store_copy(j, bb):
            return pltpu.make_async_copy(
                rows_v.at[bb],
                out_hbm.at[pl.ds(base + j * CHUNK, CHUNK)],
                ssems[bb],
            )

        def body(t, carry):
            for bb in range(2):
                j = 2 * t + bb
                @pl.when(t > 0)
                def _():
                    store_copy(j - 2, bb).wait()

                copies = []
                for q in range(QPC):
                    copies.append(pltpu.async_copy(
                        table_hbm.at[idx_v.at[j * QPC + q]],
                        rows_v.at[bb].at[pl.ds(q * G, G)],
                        gsem,
                    ))
                for cp in copies:
                    cp.wait()
                store_copy(j, bb).start()
            return carry

        lax.fori_loop(0, n_chunks // 2, body, 0)
        store_copy(n_chunks - 2, 0).wait()
        store_copy(n_chunks - 1, 1).wait()

    return gather_kernel


def kernel(input_ids, attention_mask, table):
    b, s = input_ids.shape
    _, d = table.shape
    tot = b * s
    ids = input_ids.astype(jnp.int32).reshape(NW, tot // (NW * G), G)
    out = _make_gather(tot, d)(ids, table)
    return out.reshape(b, s, d), attention_mask
